# baseline (speedup 1.0000x reference)
import jax
import jax.numpy as jnp
from jax import lax
from jax.experimental import pallas as pl
from jax.experimental.pallas import tpu as pltpu
from jax.experimental.pallas import tpu_sc as plsc

_B = 4096
_D = 128
_V = 100000

_info = plsc.get_sparse_core_info()
_NC, _NS = _info.num_cores, _info.num_subcores
_NW = _NC * _NS
_BPW = _B // _NW

_BM = 1024
_GM = _B // _BM
_BN = 2048
_GN = _V // _BN
_NBUF = 4


def _gather_body(idx_hbm, table_hbm, out_hbm, idx_v, rows_v, sem):
    wid = lax.axis_index("s") * _NC + lax.axis_index("c")
    base = wid * _BPW
    pltpu.sync_copy(idx_hbm.at[pl.ds(base, _BPW)], idx_v)
    pltpu.async_copy(table_hbm.at[idx_v], rows_v, sem).wait()
    pltpu.sync_copy(rows_v, out_hbm.at[pl.ds(base, _BPW)])


_gather = pl.kernel(
    _gather_body,
    out_type=jax.ShapeDtypeStruct((_B, _D), jnp.float32),
    mesh=plsc.VectorSubcoreMesh(core_axis_name="c", subcore_axis_name="s"),
    scratch_types=[
        pltpu.VMEM((_BPW,), jnp.int32),
        pltpu.VMEM((_BPW, _D), jnp.float32),
        pltpu.SemaphoreType.DMA,
    ],
)


def _proj_body(emb_ref, w_ref, b_ref, out_ref, buf, sems):
    n = pl.program_id(0)
    m = pl.program_id(1)
    step = n * _GM + m
    slot = lax.rem(step, _NBUF)

    def _copy(s):
        return pltpu.make_async_copy(
            buf.at[s],
            out_ref.at[pl.ds(m * _BM, _BM), pl.ds(n * _BN, _BN)],
            sems.at[s],
        )

    @pl.when(step >= _NBUF)
    def _():
        _copy(slot).wait()

    e = emb_ref[...].astype(jnp.bfloat16)
    w = w_ref[...].astype(jnp.bfloat16)
    acc = lax.dot_general(e, w, (((1,), (1,)), ((), ())),
                          preferred_element_type=jnp.float32)
    buf[slot] = acc + b_ref[...]
    _copy(slot).start()

    @pl.when(step == _GN * _GM - 1)
    def _():
        for s in range(_NBUF):
            _copy(s).wait()


_proj = pl.pallas_call(
    _proj_body,
    grid=(_GN, _GM),
    in_specs=[
        pl.BlockSpec((_BM, _D), lambda n, m: (m, 0)),
        pl.BlockSpec((_BN, _D), lambda n, m: (n, 0)),
        pl.BlockSpec((1, _BN), lambda n, m: (0, n)),
    ],
    out_specs=pl.BlockSpec(memory_space=pltpu.HBM),
    out_shape=jax.ShapeDtypeStruct((_B, _V), jnp.float32),
    scratch_shapes=[
        pltpu.VMEM((_NBUF, _BM, _BN), jnp.float32),
        pltpu.SemaphoreType.DMA((_NBUF,)),
    ],
    compiler_params=pltpu.CompilerParams(
        dimension_semantics=("arbitrary", "arbitrary"),
    ),
)


def _edge_body(logits_ref, emb_ref, w_ref, b_ref, out_ref):
    e = emb_ref[...].astype(jnp.bfloat16)
    w = w_ref[...].astype(jnp.bfloat16)
    acc = lax.dot_general(e, w, (((1,), (1,)), ((), ())),
                          preferred_element_type=jnp.float32)
    out_ref[...] = acc + b_ref[...]


_edge = pl.pallas_call(
    _edge_body,
    grid=(_GM,),
    in_specs=[
        pl.BlockSpec(memory_space=pltpu.HBM),
        pl.BlockSpec((_BM, _D), lambda m: (m, 0)),
        pl.BlockSpec((_BN, _D), lambda m: (_GN, 0)),
        pl.BlockSpec((1, _BN), lambda m: (0, _GN)),
    ],
    out_specs=pl.BlockSpec((_BM, _BN), lambda m: (m, _GN)),
    out_shape=jax.ShapeDtypeStruct((_B, _V), jnp.float32),
    input_output_aliases={0: 0},
    compiler_params=pltpu.CompilerParams(
        dimension_semantics=("arbitrary",),
    ),
)


def kernel(docs, doc_embeddings, W, b):
    emb = _gather(docs.astype(jnp.int32), doc_embeddings)
    b2 = b.reshape(1, _V)
    logits = _proj(emb, W, b2)
    return _edge(logits, emb, W, b2)


# speedup vs baseline: 3.4174x; 3.4174x over previous
import jax
import jax.numpy as jnp
from jax import lax
from jax.experimental import pallas as pl
from jax.experimental.pallas import tpu as pltpu
from jax.experimental.pallas import tpu_sc as plsc

_B = 4096
_D = 128
_V = 100000

_info = plsc.get_sparse_core_info()
_NC, _NS = _info.num_cores, _info.num_subcores
_NW = _NC * _NS
_BPW = _B // _NW

_BV = 1000
_GV = _V // _BV


def _gather_body(idx_hbm, table_hbm, out_hbm, idx_v, rows_v, sem):
    wid = lax.axis_index("s") * _NC + lax.axis_index("c")
    base = wid * _BPW
    pltpu.sync_copy(idx_hbm.at[pl.ds(base, _BPW)], idx_v)
    pltpu.async_copy(table_hbm.at[idx_v], rows_v, sem).wait()
    pltpu.sync_copy(rows_v, out_hbm.at[pl.ds(base, _BPW)])


_gather = pl.kernel(
    _gather_body,
    out_type=jax.ShapeDtypeStruct((_B, _D), jnp.float32),
    mesh=plsc.VectorSubcoreMesh(core_axis_name="c", subcore_axis_name="s"),
    scratch_types=[
        pltpu.VMEM((_BPW,), jnp.int32),
        pltpu.VMEM((_BPW, _D), jnp.float32),
        pltpu.SemaphoreType.DMA,
    ],
)


def _proj_body(w_ref, emb_ref, b_ref, out_ref):
    w = w_ref[...].astype(jnp.bfloat16)
    e = emb_ref[...].astype(jnp.bfloat16)
    acc = lax.dot_general(w, e, (((1,), (1,)), ((), ())),
                          preferred_element_type=jnp.float32)
    out_ref[...] = acc + b_ref[...]


_proj = pl.pallas_call(
    _proj_body,
    grid=(_GV,),
    in_specs=[
        pl.BlockSpec((_BV, _D), lambda v: (v, 0)),
        pl.BlockSpec((_B, _D), lambda v: (0, 0)),
        pl.BlockSpec((_BV, 1), lambda v: (v, 0)),
    ],
    out_specs=pl.BlockSpec((_BV, _B), lambda v: (v, 0)),
    out_shape=jax.ShapeDtypeStruct((_V, _B), jnp.float32),
    compiler_params=pltpu.CompilerParams(
        dimension_semantics=("arbitrary",),
    ),
)


def kernel(docs, doc_embeddings, W, b):
    emb = _gather(docs.astype(jnp.int32), doc_embeddings)
    logits_t = _proj(W, emb, b.reshape(_V, 1))
    return logits_t.T
